# baseline (device time: 191501 ns/iter reference)
import functools

import jax
import jax.numpy as jnp
from jax import lax
from jax.experimental import pallas as pl
from jax.experimental.pallas import tpu as pltpu

N_DEV = 4
TOK = 2048
HALF = 1024
D = 512
H = 1024
E = 32
E_LOC = 8
BLK = 256


def kernel(x, router_W, route_idx, expert_W):
    x = x.astype(jnp.bfloat16)
    router_W = router_W.astype(jnp.bfloat16)
    expert_W = expert_W.astype(jnp.bfloat16).reshape(E_LOC * D, H)

    def body(x_ref, rw_ref, idx_ref, ew_ref, out_ref,
             xl, xr, xo, wself, wl, wr, wo,
             co, cr, cl, h1L, h1R, r2L, r2R,
             sA_s, sA_r, sB_s, sB_r, sM1_s, sM1_r, sM2_s, sM2_r):
        my = lax.axis_index("i")
        left = lax.rem(my - 1 + N_DEV, N_DEV)
        right = lax.rem(my + 1, N_DEV)

        barrier_sem = pltpu.get_barrier_semaphore()
        for nbr in (left, right):
            pl.semaphore_signal(barrier_sem, inc=1, device_id=(nbr,),
                                device_id_type=pl.DeviceIdType.MESH)
        pl.semaphore_wait(barrier_sem, 2)

        scores = jnp.dot(x_ref[...], rw_ref[...],
                         preferred_element_type=jnp.float32)
        e0 = idx_ref[:, 0:1]
        e1 = idx_ref[:, 1:2]
        iota_e = lax.broadcasted_iota(jnp.int32, (TOK, E), 1)
        m0 = iota_e == e0
        m1 = iota_e == e1
        s0 = jnp.sum(jnp.where(m0, scores, 0.0), axis=1, keepdims=True)
        s1 = jnp.sum(jnp.where(m1, scores, 0.0), axis=1, keepdims=True)
        mx = jnp.maximum(s0, s1)
        p0 = jnp.exp(s0 - mx)
        p1 = jnp.exp(s1 - mx)
        tot = p0 + p1
        w = (jnp.where(m0, p0 / tot, 0.0)
             + jnp.where(m1, p1 / tot, 0.0)).astype(jnp.bfloat16)

        wself[0] = w[0:HALF, :]
        wself[1] = w[HALF:TOK, :]

        def rdma(src, dst, ssem, rsem, dev):
            return pltpu.make_async_remote_copy(
                src_ref=src, dst_ref=dst, send_sem=ssem, recv_sem=rsem,
                device_id=(dev,), device_id_type=pl.DeviceIdType.MESH)

        ag1 = [
            rdma(x_ref.at[0:HALF], xl.at[0], sA_s.at[0], sA_r.at[0], right),
            rdma(x_ref.at[HALF:TOK], xl.at[1], sA_s.at[1], sA_r.at[1], right),
            rdma(x_ref.at[0:HALF], xr.at[0], sA_s.at[2], sA_r.at[2], left),
            rdma(x_ref.at[HALF:TOK], xr.at[1], sA_s.at[3], sA_r.at[3], left),
            rdma(wself, wl, sA_s.at[4], sA_r.at[4], right),
            rdma(wself, wr, sA_s.at[5], sA_r.at[5], left),
        ]
        for c in ag1:
            c.start()
        for c in ag1:
            c.wait_recv()

        ag2 = [
            rdma(xl.at[0], xo.at[0], sB_s.at[0], sB_r.at[0], right),
            rdma(xr.at[1], xo.at[1], sB_s.at[1], sB_r.at[1], left),
            rdma(wl.at[0], wo.at[0], sB_s.at[2], sB_r.at[2], right),
            rdma(wr.at[1], wo.at[1], sB_s.at[3], sB_r.at[3], left),
        ]
        for c in ag2:
            c.start()
        for c in ag2:
            c.wait_recv()

        ie = lax.broadcasted_iota(jnp.int32, (E, E_LOC), 0)
        ij = lax.broadcasted_iota(jnp.int32, (E, E_LOC), 1)
        sel = (ie == my * E_LOC + ij).astype(jnp.bfloat16)

        def contrib(get_x, wh, store):
            for half in range(2):
                for t in range(0, HALF, BLK):
                    tb = pl.ds(t, BLK)
                    xq = get_x(half, t)
                    wloc = jnp.dot(wh[half, tb, :], sel,
                                   preferred_element_type=jnp.float32)
                    xs = jnp.concatenate(
                        [xq * wloc[:, e:e + 1].astype(jnp.bfloat16)
                         for e in range(E_LOC)], axis=1)
                    acc = jnp.dot(xs, ew_ref[...],
                                  preferred_element_type=jnp.float32)
                    store(half, t, acc)

        def store_bf16(dst):
            def f(half, t, acc):
                dst[half, pl.ds(t, BLK), :] = acc.astype(jnp.bfloat16)
            return f

        def read_half(buf):
            return lambda half, t: buf[half, pl.ds(t, BLK), :]

        contrib(read_half(xo), wo, store_bf16(co))
        m1 = [
            rdma(co.at[0], h1L, sM1_s.at[0], sM1_r.at[0], right),
            rdma(co.at[1], h1R, sM1_s.at[1], sM1_r.at[1], left),
        ]
        for c in m1:
            c.start()

        contrib(read_half(xr), wr, store_bf16(cr))
        contrib(read_half(xl), wl, store_bf16(cl))

        for c in m1:
            c.wait_recv()
        cr[0] = cr[0] + h1L[...]
        cl[1] = cl[1] + h1R[...]

        m2 = [
            rdma(cr, r2L, sM2_s.at[0], sM2_r.at[0], right),
            rdma(cl, r2R, sM2_s.at[1], sM2_r.at[1], left),
        ]
        for c in m2:
            c.start()

        def store_out(half, t, acc):
            out_ref[pl.ds(half * HALF + t, BLK), :] = acc
        contrib(lambda half, t: x_ref[pl.ds(half * HALF + t, BLK), :],
                wself, store_out)

        for c in m2:
            c.wait_recv()
        for half in range(2):
            for t in range(0, HALF, BLK):
                tb = pl.ds(t, BLK)
                g = pl.ds(half * HALF + t, BLK)
                out_ref[g, :] = (out_ref[g, :]
                                 + r2L[half, tb, :].astype(jnp.float32)
                                 + r2R[half, tb, :].astype(jnp.float32))

        for c in ag1 + ag2 + m1 + m2:
            c.wait_send()

        @functools.partial(pl.run_scoped, sem2=pltpu.SemaphoreType.REGULAR)
        def _(sem2):
            for nbr in (left, right):
                pl.semaphore_signal(sem2, inc=1, device_id=(nbr,),
                                    device_id_type=pl.DeviceIdType.MESH)
            pl.semaphore_wait(sem2, 2)

    half_x = (2, HALF, D)
    half_w = (2, HALF, E)
    half_h = (2, HALF, H)
    return pl.pallas_call(
        body,
        out_shape=jax.ShapeDtypeStruct((TOK, H), jnp.float32),
        in_specs=[pl.BlockSpec(memory_space=pltpu.VMEM)] * 4,
        out_specs=pl.BlockSpec(memory_space=pltpu.VMEM),
        scratch_shapes=[
            pltpu.VMEM(half_x, jnp.bfloat16),
            pltpu.VMEM(half_x, jnp.bfloat16),
            pltpu.VMEM(half_x, jnp.bfloat16),
            pltpu.VMEM(half_w, jnp.bfloat16),
            pltpu.VMEM(half_w, jnp.bfloat16),
            pltpu.VMEM(half_w, jnp.bfloat16),
            pltpu.VMEM(half_w, jnp.bfloat16),
            pltpu.VMEM(half_h, jnp.bfloat16),
            pltpu.VMEM(half_h, jnp.bfloat16),
            pltpu.VMEM(half_h, jnp.bfloat16),
            pltpu.VMEM((HALF, H), jnp.bfloat16),
            pltpu.VMEM((HALF, H), jnp.bfloat16),
            pltpu.VMEM(half_h, jnp.bfloat16),
            pltpu.VMEM(half_h, jnp.bfloat16),
            pltpu.SemaphoreType.DMA((6,)),
            pltpu.SemaphoreType.DMA((6,)),
            pltpu.SemaphoreType.DMA((4,)),
            pltpu.SemaphoreType.DMA((4,)),
            pltpu.SemaphoreType.DMA((2,)),
            pltpu.SemaphoreType.DMA((2,)),
            pltpu.SemaphoreType.DMA((2,)),
            pltpu.SemaphoreType.DMA((2,)),
        ],
        compiler_params=pltpu.CompilerParams(
            collective_id=0, vmem_limit_bytes=100 * 1024 * 1024),
    )(x, router_W, route_idx, expert_W)


# device time: 177642 ns/iter; 1.0780x vs baseline; 1.0780x over previous
import functools

import jax
import jax.numpy as jnp
from jax import lax
from jax.experimental import pallas as pl
from jax.experimental.pallas import tpu as pltpu

N_DEV = 4
TOK = 2048
HALF = 1024
D = 512
H = 1024
E = 32
E_LOC = 8
BLK = 512


def kernel(x, router_W, route_idx, expert_W):
    x = x.astype(jnp.bfloat16)
    router_W = router_W.astype(jnp.bfloat16)
    expert_W = expert_W.astype(jnp.bfloat16).reshape(E_LOC * D, H)

    def body(x_ref, rw_ref, idx_ref, ew_ref, out_ref,
             xl, xr, xo, wself, wl, wr, wo,
             co, cr, cl, h1L, h1R, r2L, r2R,
             sA_s, sA_r, sB_s, sB_r, sM1_s, sM1_r, sM2_s, sM2_r):
        my = lax.axis_index("i")
        left = lax.rem(my - 1 + N_DEV, N_DEV)
        right = lax.rem(my + 1, N_DEV)

        barrier_sem = pltpu.get_barrier_semaphore()
        for nbr in (left, right):
            pl.semaphore_signal(barrier_sem, inc=1, device_id=(nbr,),
                                device_id_type=pl.DeviceIdType.MESH)
        pl.semaphore_wait(barrier_sem, 2)

        scores = jnp.dot(x_ref[...], rw_ref[...],
                         preferred_element_type=jnp.float32)
        e0 = idx_ref[:, 0:1]
        e1 = idx_ref[:, 1:2]
        iota_e = lax.broadcasted_iota(jnp.int32, (TOK, E), 1)
        m0 = iota_e == e0
        m1 = iota_e == e1
        s0 = jnp.sum(jnp.where(m0, scores, 0.0), axis=1, keepdims=True)
        s1 = jnp.sum(jnp.where(m1, scores, 0.0), axis=1, keepdims=True)
        mx = jnp.maximum(s0, s1)
        p0 = jnp.exp(s0 - mx)
        p1 = jnp.exp(s1 - mx)
        tot = p0 + p1
        w = (jnp.where(m0, p0 / tot, 0.0)
             + jnp.where(m1, p1 / tot, 0.0)).astype(jnp.bfloat16)

        wself[0] = w[0:HALF, :]
        wself[1] = w[HALF:TOK, :]

        def rdma(src, dst, ssem, rsem, dev):
            return pltpu.make_async_remote_copy(
                src_ref=src, dst_ref=dst, send_sem=ssem, recv_sem=rsem,
                device_id=(dev,), device_id_type=pl.DeviceIdType.MESH)

        ag1 = [
            rdma(x_ref.at[0:HALF], xl.at[0], sA_s.at[0], sA_r.at[0], right),
            rdma(x_ref.at[HALF:TOK], xl.at[1], sA_s.at[1], sA_r.at[1], right),
            rdma(x_ref.at[0:HALF], xr.at[0], sA_s.at[2], sA_r.at[2], left),
            rdma(x_ref.at[HALF:TOK], xr.at[1], sA_s.at[3], sA_r.at[3], left),
            rdma(wself, wl, sA_s.at[4], sA_r.at[4], right),
            rdma(wself, wr, sA_s.at[5], sA_r.at[5], left),
        ]
        for c in ag1:
            c.start()
        for c in ag1:
            c.wait_recv()

        ag2 = [
            rdma(xl.at[0], xo.at[0], sB_s.at[0], sB_r.at[0], right),
            rdma(xr.at[1], xo.at[1], sB_s.at[1], sB_r.at[1], left),
            rdma(wl.at[0], wo.at[0], sB_s.at[2], sB_r.at[2], right),
            rdma(wr.at[1], wo.at[1], sB_s.at[3], sB_r.at[3], left),
        ]
        for c in ag2:
            c.start()

        ie = lax.broadcasted_iota(jnp.int32, (E, E_LOC), 0)
        ij = lax.broadcasted_iota(jnp.int32, (E, E_LOC), 1)
        sel = (ie == my * E_LOC + ij).astype(jnp.bfloat16)

        def contrib(get_x, wh, store):
            for half in range(2):
                for t in range(0, HALF, BLK):
                    tb = pl.ds(t, BLK)
                    xq = get_x(half, t)
                    wloc = jnp.dot(wh[half, tb, :], sel,
                                   preferred_element_type=jnp.float32)
                    acc = jnp.zeros((BLK, H), jnp.float32)
                    for e in range(E_LOC):
                        gate = wloc[:, e:e + 1].astype(jnp.bfloat16)
                        acc = acc + jnp.dot(
                            xq * gate, ew_ref[pl.ds(e * D, D), :],
                            preferred_element_type=jnp.float32)
                    store(half, t, acc)

        def store_bf16(dst):
            def f(half, t, acc):
                dst[half, pl.ds(t, BLK), :] = acc.astype(jnp.bfloat16)
            return f

        def read_half(buf):
            return lambda half, t: buf[half, pl.ds(t, BLK), :]

        contrib(read_half(xr), wr, store_bf16(cr))

        for c in ag2:
            c.wait_recv()

        contrib(read_half(xo), wo, store_bf16(co))
        m1 = [
            rdma(co.at[0], h1L, sM1_s.at[0], sM1_r.at[0], right),
            rdma(co.at[1], h1R, sM1_s.at[1], sM1_r.at[1], left),
        ]
        for c in m1:
            c.start()

        contrib(read_half(xl), wl, store_bf16(cl))

        for c in m1:
            c.wait_recv()
        cr[0] = cr[0] + h1L[...]
        cl[1] = cl[1] + h1R[...]

        m2 = [
            rdma(cr, r2L, sM2_s.at[0], sM2_r.at[0], right),
            rdma(cl, r2R, sM2_s.at[1], sM2_r.at[1], left),
        ]
        for c in m2:
            c.start()

        def store_out(half, t, acc):
            out_ref[pl.ds(half * HALF + t, BLK), :] = acc
        contrib(lambda half, t: x_ref[pl.ds(half * HALF + t, BLK), :],
                wself, store_out)

        for c in m2:
            c.wait_recv()
        for half in range(2):
            for t in range(0, HALF, BLK):
                tb = pl.ds(t, BLK)
                g = pl.ds(half * HALF + t, BLK)
                out_ref[g, :] = (out_ref[g, :]
                                 + r2L[half, tb, :].astype(jnp.float32)
                                 + r2R[half, tb, :].astype(jnp.float32))

        for c in ag1 + ag2 + m1 + m2:
            c.wait_send()

        @functools.partial(pl.run_scoped, sem2=pltpu.SemaphoreType.REGULAR)
        def _(sem2):
            for nbr in (left, right):
                pl.semaphore_signal(sem2, inc=1, device_id=(nbr,),
                                    device_id_type=pl.DeviceIdType.MESH)
            pl.semaphore_wait(sem2, 2)

    half_x = (2, HALF, D)
    half_w = (2, HALF, E)
    half_h = (2, HALF, H)
    return pl.pallas_call(
        body,
        out_shape=jax.ShapeDtypeStruct((TOK, H), jnp.float32),
        in_specs=[pl.BlockSpec(memory_space=pltpu.VMEM)] * 4,
        out_specs=pl.BlockSpec(memory_space=pltpu.VMEM),
        scratch_shapes=[
            pltpu.VMEM(half_x, jnp.bfloat16),
            pltpu.VMEM(half_x, jnp.bfloat16),
            pltpu.VMEM(half_x, jnp.bfloat16),
            pltpu.VMEM(half_w, jnp.bfloat16),
            pltpu.VMEM(half_w, jnp.bfloat16),
            pltpu.VMEM(half_w, jnp.bfloat16),
            pltpu.VMEM(half_w, jnp.bfloat16),
            pltpu.VMEM(half_h, jnp.bfloat16),
            pltpu.VMEM(half_h, jnp.bfloat16),
            pltpu.VMEM(half_h, jnp.bfloat16),
            pltpu.VMEM((HALF, H), jnp.bfloat16),
            pltpu.VMEM((HALF, H), jnp.bfloat16),
            pltpu.VMEM(half_h, jnp.bfloat16),
            pltpu.VMEM(half_h, jnp.bfloat16),
            pltpu.SemaphoreType.DMA((6,)),
            pltpu.SemaphoreType.DMA((6,)),
            pltpu.SemaphoreType.DMA((4,)),
            pltpu.SemaphoreType.DMA((4,)),
            pltpu.SemaphoreType.DMA((2,)),
            pltpu.SemaphoreType.DMA((2,)),
            pltpu.SemaphoreType.DMA((2,)),
            pltpu.SemaphoreType.DMA((2,)),
        ],
        compiler_params=pltpu.CompilerParams(
            collective_id=0, vmem_limit_bytes=100 * 1024 * 1024),
    )(x, router_W, route_idx, expert_W)
